# Initial kernel scaffold; baseline (speedup 1.0000x reference)
import jax
import jax.numpy as jnp
from jax.experimental import pallas as pl


def kernel(x, edge_index, edge_attr, Wi, bi, Wq, bq, Wk, bk, Wv, bv, We, be, Ws, bs, Wo, bo):
    raise NotImplementedError("write your pallas kernel here")


# speedup vs baseline: 13.7816x; 13.7816x over previous
import functools
import math

import jax
import jax.numpy as jnp
from jax import lax
from jax.experimental import pallas as pl
from jax.experimental.pallas import tpu as pltpu
from jax.experimental.pallas import tpu_sc as plsc

_NC = 2
_NS = 16
_CB = 128
_HEADS = 8

_SC_PARAMS = pltpu.CompilerParams(use_tc_tiling_on_sc=False,
                                  needs_layout_passes=False)

_f32 = jnp.float32
_i32 = jnp.int32


def _mm_in_body(x_ref, w_ref, b_ref, o_ref):
    o_ref[...] = jnp.dot(x_ref[...], w_ref[...],
                         preferred_element_type=_f32) + b_ref[...]


def _proj_h_body(h_ref, w_ref, b_ref, q_ref, k_ref, v0_ref, v1_ref, s_ref):
    p = jnp.dot(h_ref[...], w_ref[...], preferred_element_type=_f32) + b_ref[...]
    hc = q_ref.shape[1]
    q_ref[...] = p[:, :hc]
    k_ref[...] = p[:, hc:2 * hc]
    v0_ref[...] = p[:, 2 * hc:2 * hc + hc // 2]
    v1_ref[...] = p[:, 2 * hc + hc // 2:3 * hc]
    s_ref[...] = p[:, 3 * hc:]


def _proj_agg_body(a0_ref, a1_ref, sin_ref, w_ref, b_ref,
                   q_ref, k_ref, v0_ref, v1_ref, s_ref):
    h = jnp.concatenate([a0_ref[0] + a0_ref[1], a1_ref[0] + a1_ref[1]],
                        axis=1) + sin_ref[...]
    p = jnp.dot(h, w_ref[...], preferred_element_type=_f32) + b_ref[...]
    hc = q_ref.shape[1]
    q_ref[...] = p[:, :hc]
    k_ref[...] = p[:, hc:2 * hc]
    v0_ref[...] = p[:, 2 * hc:2 * hc + hc // 2]
    v1_ref[...] = p[:, 2 * hc + hc // 2:3 * hc]
    s_ref[...] = p[:, 3 * hc:]


def _edge_body(ea_ref, w_ref, b_ref, e0_ref, e1_ref):
    p = jnp.dot(ea_ref[...], w_ref[...], preferred_element_type=_f32) + b_ref[...]
    hf = e0_ref.shape[1]
    e0_ref[...] = p[:, :hf]
    e1_ref[...] = p[:, hf:]


def _final_body(a0_ref, a1_ref, sin_ref, w_ref, b_ref, o_ref):
    h = jnp.concatenate([a0_ref[0] + a0_ref[1], a1_ref[0] + a1_ref[1]],
                        axis=1) + sin_ref[...]
    o_ref[...] = jax.nn.sigmoid(
        jnp.dot(h, w_ref[...], preferred_element_type=_f32) + b_ref[...])


def _row_spec(br, w):
    return pl.BlockSpec((br, w), lambda i: (i, 0))


def _fix_spec(shape):
    nd = len(shape)
    return pl.BlockSpec(shape, lambda i: (0,) * nd)


def _pair_spec(br, w):
    return pl.BlockSpec((2, br, w), lambda i: (0, i, 0))


def _pass1_body(scale, nblk, q_h, k_h, ep0_h, ep1_h, src_h, dst_h, z_h,
                ex_h, den_h,
                idx_s, idx_d, qb, kb, eb0, eb1, exb, den_sh):
    c = lax.axis_index("c")
    s = lax.axis_index("s")
    wid = s * _NC + c
    n = den_sh.shape[0]
    rp = n // _NS
    r0 = s * rp
    pltpu.sync_copy(z_h.at[pl.ds(r0, rp)], den_sh.at[pl.ds(r0, rp)])
    zv = jnp.zeros((16,), _f32)

    def _zb(i, _):
        exb[i] = zv
        return 0
    lax.fori_loop(0, _CB, _zb, 0)
    plsc.subcore_barrier()

    iota = lax.iota(_i32, 16)
    nloop = (nblk + _NC * _NS - 1) // (_NC * _NS)

    def _blk(i, _):
        blk = wid + i * (_NC * _NS)

        @pl.when(blk < nblk)
        def _():
            e0 = blk * _CB
            pltpu.sync_copy(src_h.at[pl.ds(e0, _CB)], idx_s)
            pltpu.sync_copy(dst_h.at[pl.ds(e0, _CB)], idx_d)
            pltpu.sync_copy(q_h.at[idx_d], qb)
            pltpu.sync_copy(k_h.at[idx_s], kb)
            pltpu.sync_copy(ep0_h.at[pl.ds(e0, _CB)], eb0)
            pltpu.sync_copy(ep1_h.at[pl.ds(e0, _CB)], eb1)

            def _grp(g, _2):
                erow = iota + g * 16
                acc = [jnp.zeros((16,), _f32)] * _HEADS
                hw = qb.shape[1] // _HEADS
                for cf in range(qb.shape[1]):
                    col = jnp.full((16,), cf, _i32)
                    qv = plsc.load_gather(qb, [erow, col])
                    kv = plsc.load_gather(kb, [erow, col])
                    if cf < qb.shape[1] // 2:
                        ev = plsc.load_gather(eb0, [erow, col])
                    else:
                        ev = plsc.load_gather(
                            eb1, [erow, jnp.full((16,), cf - qb.shape[1] // 2, _i32)])
                    hh = cf // hw
                    acc[hh] = acc[hh] + qv * (kv + ev)
                for hh in range(_HEADS):
                    exv = jnp.exp(acc[hh] * scale)
                    plsc.store_scatter(exb, [erow, jnp.full((16,), hh, _i32)], exv)
                return 0
            lax.fori_loop(0, _CB // 16, _grp, 0)
            pltpu.sync_copy(exb, ex_h.at[pl.ds(e0, _CB)])
            pltpu.sync_copy(exb, den_sh.at[idx_d], add=True)
        return 0
    lax.fori_loop(0, nloop, _blk, 0)

    plsc.subcore_barrier()
    pltpu.sync_copy(den_sh.at[pl.ds(r0, rp)], den_h.at[c, pl.ds(r0, rp)])


def _pass2_body(hh, nblk, v_h, ep_h, ex_h, den0_h, den1_h, src_h, dst_h, z_h,
                agg_h,
                idx_s, idx_d, vb, eb, exb, d0b, d1b, mb, agg_sh):
    c = lax.axis_index("c")
    s = lax.axis_index("s")
    wid = s * _NC + c
    n = agg_sh.shape[0]
    rp = n // _NS
    r0 = s * rp
    pltpu.sync_copy(z_h.at[pl.ds(r0, rp)], agg_sh.at[pl.ds(r0, rp)])
    plsc.subcore_barrier()

    iota = lax.iota(_i32, 16)
    sh3 = lax.shift_right_logical(iota, 3)
    col0 = jnp.full((16,), 4 * hh, _i32) + sh3
    col1 = jnp.full((16,), 4 * hh + 2, _i32) + sh3
    nloop = (nblk + _NC * _NS - 1) // (_NC * _NS)

    def _blk(i, _):
        blk = wid + i * (_NC * _NS)

        @pl.when(blk < nblk)
        def _():
            e0 = blk * _CB
            pltpu.sync_copy(src_h.at[pl.ds(e0, _CB)], idx_s)
            pltpu.sync_copy(dst_h.at[pl.ds(e0, _CB)], idx_d)
            pltpu.sync_copy(v_h.at[idx_s], vb)
            pltpu.sync_copy(ep_h.at[pl.ds(e0, _CB)], eb)
            pltpu.sync_copy(ex_h.at[pl.ds(e0, _CB)], exb)
            pltpu.sync_copy(den0_h.at[idx_d], d0b)
            pltpu.sync_copy(den1_h.at[idx_d], d1b)

            def _edge(e, _2):
                se = jnp.full((16,), e, _i32)
                ex0 = plsc.load_gather(exb, [se, col0])
                da0 = plsc.load_gather(d0b, [se, col0])
                db0 = plsc.load_gather(d1b, [se, col0])
                a0 = ex0 / (da0 + db0 + 1e-16)
                ex1 = plsc.load_gather(exb, [se, col1])
                da1 = plsc.load_gather(d0b, [se, col1])
                db1 = plsc.load_gather(d1b, [se, col1])
                a1 = ex1 / (da1 + db1 + 1e-16)
                mb[e, pl.ds(0, 16)] = (vb[e, pl.ds(0, 16)] + eb[e, pl.ds(0, 16)]) * a0
                mb[e, pl.ds(16, 16)] = (vb[e, pl.ds(16, 16)] + eb[e, pl.ds(16, 16)]) * a1
                return 0
            lax.fori_loop(0, _CB, _edge, 0)
            pltpu.sync_copy(mb, agg_sh.at[idx_d], add=True)
        return 0
    lax.fori_loop(0, nloop, _blk, 0)

    plsc.subcore_barrier()
    pltpu.sync_copy(agg_sh.at[pl.ds(r0, rp)], agg_h.at[c, pl.ds(r0, rp)])


def _build_pass1(n_pad, e, hc, scale):
    nblk = e // _CB
    mesh = plsc.VectorSubcoreMesh(core_axis_name="c", subcore_axis_name="s")
    return pl.kernel(
        functools.partial(_pass1_body, scale, nblk),
        compiler_params=_SC_PARAMS,
        out_type=(jax.ShapeDtypeStruct((e, 16), _f32),
                  jax.ShapeDtypeStruct((2, n_pad, 16), _f32)),
        mesh=mesh,
        scratch_types=[
            pltpu.VMEM((_CB,), _i32),
            pltpu.VMEM((_CB,), _i32),
            pltpu.VMEM((_CB, hc), _f32),
            pltpu.VMEM((_CB, hc), _f32),
            pltpu.VMEM((_CB, hc // 2), _f32),
            pltpu.VMEM((_CB, hc // 2), _f32),
            pltpu.VMEM((_CB, 16), _f32),
            pltpu.VMEM_SHARED((n_pad, 16), _f32),
        ],
    )


def _build_pass2(n_pad, e, hc, hh):
    nblk = e // _CB
    hf = hc // 2
    mesh = plsc.VectorSubcoreMesh(core_axis_name="c", subcore_axis_name="s")
    return pl.kernel(
        functools.partial(_pass2_body, hh, nblk),
        compiler_params=_SC_PARAMS,
        out_type=jax.ShapeDtypeStruct((2, n_pad, hf), _f32),
        mesh=mesh,
        scratch_types=[
            pltpu.VMEM((_CB,), _i32),
            pltpu.VMEM((_CB,), _i32),
            pltpu.VMEM((_CB, hf), _f32),
            pltpu.VMEM((_CB, hf), _f32),
            pltpu.VMEM((_CB, 16), _f32),
            pltpu.VMEM((_CB, 16), _f32),
            pltpu.VMEM((_CB, 16), _f32),
            pltpu.VMEM((_CB, hf), _f32),
            pltpu.VMEM_SHARED((n_pad, hf), _f32),
        ],
    )


def kernel(x, edge_index, edge_attr, Wi, bi, Wq, bq, Wk, bk, Wv, bv,
           We, be, Ws, bs, Wo, bo):
    n, d_in = x.shape
    e = edge_index.shape[1]
    hc = Wi.shape[0]
    d_e = edge_attr.shape[1]
    n_layers = Wq.shape[0]
    hid = hc // _HEADS
    scale = 1.0 / math.sqrt(hid)
    br = 2000
    br_e = 4000

    src = edge_index[0]
    dst = edge_index[1]

    mm_in = pl.pallas_call(
        _mm_in_body,
        grid=(n // br,),
        in_specs=[_row_spec(br, d_in), _fix_spec((d_in, hc)), _fix_spec((1, hc))],
        out_specs=_row_spec(br, hc),
        out_shape=jax.ShapeDtypeStruct((n, hc), _f32),
    )
    proj_outs = dict(
        out_specs=[_row_spec(br, hc), _row_spec(br, hc),
                   _row_spec(br, hc // 2), _row_spec(br, hc // 2),
                   _row_spec(br, hc)],
        out_shape=[jax.ShapeDtypeStruct((n, hc), _f32),
                   jax.ShapeDtypeStruct((n, hc), _f32),
                   jax.ShapeDtypeStruct((n, hc // 2), _f32),
                   jax.ShapeDtypeStruct((n, hc // 2), _f32),
                   jax.ShapeDtypeStruct((n, hc), _f32)],
    )
    proj_h = pl.pallas_call(
        _proj_h_body,
        grid=(n // br,),
        in_specs=[_row_spec(br, hc), _fix_spec((hc, 4 * hc)), _fix_spec((1, 4 * hc))],
        **proj_outs,
    )
    proj_agg = pl.pallas_call(
        _proj_agg_body,
        grid=(n // br,),
        in_specs=[_pair_spec(br, hc // 2), _pair_spec(br, hc // 2),
                  _row_spec(br, hc), _fix_spec((hc, 4 * hc)), _fix_spec((1, 4 * hc))],
        **proj_outs,
    )
    edge_proj = pl.pallas_call(
        _edge_body,
        grid=(e // br_e,),
        in_specs=[_row_spec(br_e, d_e), _fix_spec((d_e, hc)), _fix_spec((1, hc))],
        out_specs=[_row_spec(br_e, hc // 2), _row_spec(br_e, hc // 2)],
        out_shape=[jax.ShapeDtypeStruct((e, hc // 2), _f32),
                   jax.ShapeDtypeStruct((e, hc // 2), _f32)],
    )
    final = pl.pallas_call(
        _final_body,
        grid=(n // br,),
        in_specs=[_pair_spec(br, hc // 2), _pair_spec(br, hc // 2),
                  _row_spec(br, hc), _fix_spec((hc, 1)), _fix_spec((1, 1))],
        out_specs=_row_spec(br, 1),
        out_shape=jax.ShapeDtypeStruct((n, 1), _f32),
    )

    n_pad = -(-n // 128) * 128
    p1 = _build_pass1(n_pad, e, hc, scale)
    p2 = [_build_pass2(n_pad, e, hc, 0), _build_pass2(n_pad, e, hc, 1)]

    z16 = jnp.zeros((n_pad, 16), _f32)
    z32 = jnp.zeros((n_pad, 32), _f32)

    h = mm_in(x, Wi.T, bi[None, :])
    agg0 = agg1 = skip = None
    for l in range(n_layers):
        wcat = jnp.concatenate([Wq[l].T, Wk[l].T, Wv[l].T, Ws[l].T], axis=1)
        bcat = jnp.concatenate([bq[l], bk[l], bv[l], bs[l]])[None, :]
        if l == 0:
            Q, K, V0, V1, skip = proj_h(h, wcat, bcat)
        else:
            Q, K, V0, V1, skip = proj_agg(agg0, agg1, skip, wcat, bcat)
        ep0, ep1 = edge_proj(edge_attr, We[l].T, be[l][None, :])
        ex, den = p1(Q, K, ep0, ep1, src, dst, z16)
        den0, den1 = den[0], den[1]
        agg0 = p2[0](V0, ep0, ex, den0, den1, src, dst, z32)
        agg1 = p2[1](V1, ep1, ex, den0, den1, src, dst, z32)
    return final(agg0, agg1, skip, Wo.T, bo[None, :])


# speedup vs baseline: 15.1082x; 1.0963x over previous
import functools
import math

import jax
import jax.numpy as jnp
from jax import lax
from jax.experimental import pallas as pl
from jax.experimental.pallas import tpu as pltpu
from jax.experimental.pallas import tpu_sc as plsc

_NC = 2
_NS = 16
_NW = _NC * _NS
_CB = 128
_HEADS = 8

_SC_PARAMS = pltpu.CompilerParams(use_tc_tiling_on_sc=False,
                                  needs_layout_passes=False)

_f32 = jnp.float32
_i32 = jnp.int32


def _mm_in_body(x_ref, w_ref, b_ref, o_ref):
    o_ref[...] = jnp.dot(x_ref[...], w_ref[...],
                         preferred_element_type=_f32) + b_ref[...]


def _proj_h_body(h_ref, w_ref, b_ref, q_ref, k_ref, v0_ref, v1_ref, s_ref):
    p = jnp.dot(h_ref[...], w_ref[...], preferred_element_type=_f32) + b_ref[...]
    hc = q_ref.shape[1]
    q_ref[...] = p[:, :hc]
    k_ref[...] = p[:, hc:2 * hc]
    v0_ref[...] = p[:, 2 * hc:2 * hc + hc // 2]
    v1_ref[...] = p[:, 2 * hc + hc // 2:3 * hc]
    s_ref[...] = p[:, 3 * hc:]


def _proj_agg_body(a0_ref, a1_ref, sin_ref, w_ref, b_ref,
                   q_ref, k_ref, v0_ref, v1_ref, s_ref):
    h = jnp.concatenate([a0_ref[0] + a0_ref[1], a1_ref[0] + a1_ref[1]],
                        axis=1) + sin_ref[...]
    p = jnp.dot(h, w_ref[...], preferred_element_type=_f32) + b_ref[...]
    hc = q_ref.shape[1]
    q_ref[...] = p[:, :hc]
    k_ref[...] = p[:, hc:2 * hc]
    v0_ref[...] = p[:, 2 * hc:2 * hc + hc // 2]
    v1_ref[...] = p[:, 2 * hc + hc // 2:3 * hc]
    s_ref[...] = p[:, 3 * hc:]


def _edge_body(ea_ref, w_ref, b_ref, e0_ref, e1_ref):
    p = jnp.dot(ea_ref[...], w_ref[...], preferred_element_type=_f32) + b_ref[...]
    hf = e0_ref.shape[1]
    e0_ref[...] = p[:, :hf]
    e1_ref[...] = p[:, hf:]


def _inv_body(den_ref, o_ref):
    o_ref[...] = 1.0 / (den_ref[0] + den_ref[1] + 1e-16)


def _final_body(a0_ref, a1_ref, sin_ref, w_ref, b_ref, o_ref):
    h = jnp.concatenate([a0_ref[0] + a0_ref[1], a1_ref[0] + a1_ref[1]],
                        axis=1) + sin_ref[...]
    o_ref[...] = jax.nn.sigmoid(
        jnp.dot(h, w_ref[...], preferred_element_type=_f32) + b_ref[...])


def _row_spec(br, w):
    return pl.BlockSpec((br, w), lambda i: (i, 0))


def _fix_spec(shape):
    nd = len(shape)
    return pl.BlockSpec(shape, lambda i: (0,) * nd)


def _pair_spec(br, w):
    return pl.BlockSpec((2, br, w), lambda i: (0, i, 0))


def _pass1_body(scale, nblk, q_h, k_h, ep0_h, ep1_h, src_h, dst_h, z_h,
                ex_h, den_h,
                idx_s, idx_d, qb, kb, eb0, eb1, exb, den_sh):
    c = lax.axis_index("c")
    s = lax.axis_index("s")
    wid = s * _NC + c
    n = den_sh.shape[0]
    rp = n // _NS
    r0 = s * rp
    pltpu.sync_copy(z_h.at[pl.ds(r0, rp)], den_sh.at[pl.ds(r0, rp)])
    zv = jnp.zeros((16,), _f32)

    def _zb(i, _):
        exb[i] = zv
        return 0
    lax.fori_loop(0, _CB, _zb, 0)
    plsc.subcore_barrier()

    iota = lax.iota(_i32, 16)
    hc = qb.shape[1]
    nloop = (nblk + _NW - 1) // _NW

    def _blk(i, _):
        blk = wid + i * _NW

        @pl.when(blk < nblk)
        def _():
            e0 = blk * _CB
            pltpu.sync_copy(src_h.at[pl.ds(e0, _CB)], idx_s)
            pltpu.sync_copy(dst_h.at[pl.ds(e0, _CB)], idx_d)
            pltpu.sync_copy(q_h.at[idx_d], qb)
            pltpu.sync_copy(k_h.at[idx_s], kb)
            pltpu.sync_copy(ep0_h.at[pl.ds(e0, _CB)], eb0)
            pltpu.sync_copy(ep1_h.at[pl.ds(e0, _CB)], eb1)

            def _grp(g, _2):
                erow = iota + g * 16
                acc = [jnp.zeros((16,), _f32)] * _HEADS
                hw = hc // _HEADS
                for cf in range(hc):
                    col = jnp.full((16,), cf, _i32)
                    qv = plsc.load_gather(qb, [erow, col])
                    kv = plsc.load_gather(kb, [erow, col])
                    if cf < hc // 2:
                        ev = plsc.load_gather(eb0, [erow, col])
                    else:
                        ev = plsc.load_gather(
                            eb1, [erow, jnp.full((16,), cf - hc // 2, _i32)])
                    hh = cf // hw
                    acc[hh] = acc[hh] + qv * (kv + ev)
                for hh in range(_HEADS):
                    exv = jnp.exp(acc[hh] * scale)
                    plsc.store_scatter(exb, [erow, jnp.full((16,), hh, _i32)], exv)
                return 0
            lax.fori_loop(0, _CB // 16, _grp, 0)

            pltpu.sync_copy(exb, ex_h.at[pl.ds(e0, _CB)])
            pltpu.sync_copy(exb, den_sh.at[idx_d], add=True)
        return 0
    lax.fori_loop(0, nloop, _blk, 0)

    plsc.subcore_barrier()
    pltpu.sync_copy(den_sh.at[pl.ds(r0, rp)], den_h.at[c, pl.ds(r0, rp)])


def _pass2_body(hh, nblk, v_h, ep_h, ex_h, inv_h, src_h, dst_h, z_h,
                agg_h,
                idx_s, idx_d, vb, eb, exb, ivb, mb, agg_sh):
    c = lax.axis_index("c")
    s = lax.axis_index("s")
    wid = s * _NC + c
    n = agg_sh.shape[0]
    rp = n // _NS
    r0 = s * rp
    pltpu.sync_copy(z_h.at[pl.ds(r0, rp)], agg_sh.at[pl.ds(r0, rp)])
    plsc.subcore_barrier()

    iota = lax.iota(_i32, 16)
    sh3 = lax.shift_right_logical(iota, 3)
    col0 = jnp.full((16,), 4 * hh, _i32) + sh3
    col1 = jnp.full((16,), 4 * hh + 2, _i32) + sh3
    nloop = (nblk + _NW - 1) // _NW

    def _blk(i, _):
        blk = wid + i * _NW

        @pl.when(blk < nblk)
        def _():
            e0 = blk * _CB
            pltpu.sync_copy(src_h.at[pl.ds(e0, _CB)], idx_s)
            pltpu.sync_copy(dst_h.at[pl.ds(e0, _CB)], idx_d)
            pltpu.sync_copy(v_h.at[idx_s], vb)
            pltpu.sync_copy(inv_h.at[idx_d], ivb)
            pltpu.sync_copy(ep_h.at[pl.ds(e0, _CB)], eb)
            pltpu.sync_copy(ex_h.at[pl.ds(e0, _CB)], exb)

            def _edge(e, _2):
                se = jnp.full((16,), e, _i32)
                a0 = (plsc.load_gather(exb, [se, col0]) *
                      plsc.load_gather(ivb, [se, col0]))
                a1 = (plsc.load_gather(exb, [se, col1]) *
                      plsc.load_gather(ivb, [se, col1]))
                mb[e, pl.ds(0, 16)] = (vb[e, pl.ds(0, 16)] +
                                       eb[e, pl.ds(0, 16)]) * a0
                mb[e, pl.ds(16, 16)] = (vb[e, pl.ds(16, 16)] +
                                        eb[e, pl.ds(16, 16)]) * a1
                return 0
            lax.fori_loop(0, _CB, _edge, 0)
            pltpu.sync_copy(mb, agg_sh.at[idx_d], add=True)
        return 0
    lax.fori_loop(0, nloop, _blk, 0)

    plsc.subcore_barrier()
    pltpu.sync_copy(agg_sh.at[pl.ds(r0, rp)], agg_h.at[c, pl.ds(r0, rp)])


def _build_pass1(n_pad, e, hc, scale):
    nblk = e // _CB
    mesh = plsc.VectorSubcoreMesh(core_axis_name="c", subcore_axis_name="s")
    return pl.kernel(
        functools.partial(_pass1_body, scale, nblk),
        compiler_params=_SC_PARAMS,
        out_type=(jax.ShapeDtypeStruct((e, 16), _f32),
                  jax.ShapeDtypeStruct((2, n_pad, 16), _f32)),
        mesh=mesh,
        scratch_types=[
            pltpu.VMEM((_CB,), _i32),
            pltpu.VMEM((_CB,), _i32),
            pltpu.VMEM((_CB, hc), _f32),
            pltpu.VMEM((_CB, hc), _f32),
            pltpu.VMEM((_CB, hc // 2), _f32),
            pltpu.VMEM((_CB, hc // 2), _f32),
            pltpu.VMEM((_CB, 16), _f32),
            pltpu.VMEM_SHARED((n_pad, 16), _f32),
        ],
    )


def _build_pass2(n_pad, e, hc, hh):
    nblk = e // _CB
    hf = hc // 2
    mesh = plsc.VectorSubcoreMesh(core_axis_name="c", subcore_axis_name="s")
    return pl.kernel(
        functools.partial(_pass2_body, hh, nblk),
        compiler_params=_SC_PARAMS,
        out_type=jax.ShapeDtypeStruct((2, n_pad, hf), _f32),
        mesh=mesh,
        scratch_types=[
            pltpu.VMEM((_CB,), _i32),
            pltpu.VMEM((_CB,), _i32),
            pltpu.VMEM((_CB, hf), _f32),
            pltpu.VMEM((_CB, hf), _f32),
            pltpu.VMEM((_CB, 16), _f32),
            pltpu.VMEM((_CB, 16), _f32),
            pltpu.VMEM((_CB, hf), _f32),
            pltpu.VMEM_SHARED((n_pad, hf), _f32),
        ],
    )


def kernel(x, edge_index, edge_attr, Wi, bi, Wq, bq, Wk, bk, Wv, bv,
           We, be, Ws, bs, Wo, bo):
    n, d_in = x.shape
    e = edge_index.shape[1]
    hc = Wi.shape[0]
    d_e = edge_attr.shape[1]
    n_layers = Wq.shape[0]
    hid = hc // _HEADS
    scale = 1.0 / math.sqrt(hid)
    br = 2000
    br_e = 4000

    src = edge_index[0]
    dst = edge_index[1]

    mm_in = pl.pallas_call(
        _mm_in_body,
        grid=(n // br,),
        in_specs=[_row_spec(br, d_in), _fix_spec((d_in, hc)), _fix_spec((1, hc))],
        out_specs=_row_spec(br, hc),
        out_shape=jax.ShapeDtypeStruct((n, hc), _f32),
    )
    proj_outs = dict(
        out_specs=[_row_spec(br, hc), _row_spec(br, hc),
                   _row_spec(br, hc // 2), _row_spec(br, hc // 2),
                   _row_spec(br, hc)],
        out_shape=[jax.ShapeDtypeStruct((n, hc), _f32),
                   jax.ShapeDtypeStruct((n, hc), _f32),
                   jax.ShapeDtypeStruct((n, hc // 2), _f32),
                   jax.ShapeDtypeStruct((n, hc // 2), _f32),
                   jax.ShapeDtypeStruct((n, hc), _f32)],
    )
    proj_h = pl.pallas_call(
        _proj_h_body,
        grid=(n // br,),
        in_specs=[_row_spec(br, hc), _fix_spec((hc, 4 * hc)), _fix_spec((1, 4 * hc))],
        **proj_outs,
    )
    proj_agg = pl.pallas_call(
        _proj_agg_body,
        grid=(n // br,),
        in_specs=[_pair_spec(br, hc // 2), _pair_spec(br, hc // 2),
                  _row_spec(br, hc), _fix_spec((hc, 4 * hc)), _fix_spec((1, 4 * hc))],
        **proj_outs,
    )
    edge_proj = pl.pallas_call(
        _edge_body,
        grid=(e // br_e,),
        in_specs=[_row_spec(br_e, d_e), _fix_spec((d_e, hc)), _fix_spec((1, hc))],
        out_specs=[_row_spec(br_e, hc // 2), _row_spec(br_e, hc // 2)],
        out_shape=[jax.ShapeDtypeStruct((e, hc // 2), _f32),
                   jax.ShapeDtypeStruct((e, hc // 2), _f32)],
    )
    final = pl.pallas_call(
        _final_body,
        grid=(n // br,),
        in_specs=[_pair_spec(br, hc // 2), _pair_spec(br, hc // 2),
                  _row_spec(br, hc), _fix_spec((hc, 1)), _fix_spec((1, 1))],
        out_specs=_row_spec(br, 1),
        out_shape=jax.ShapeDtypeStruct((n, 1), _f32),
    )

    n_pad = -(-n // 128) * 128
    br_i = n_pad // 8
    inv_den = pl.pallas_call(
        _inv_body,
        grid=(n_pad // br_i,),
        in_specs=[_pair_spec(br_i, 16)],
        out_specs=_row_spec(br_i, 16),
        out_shape=jax.ShapeDtypeStruct((n_pad, 16), _f32),
    )

    p1 = _build_pass1(n_pad, e, hc, scale)
    p2 = [_build_pass2(n_pad, e, hc, 0), _build_pass2(n_pad, e, hc, 1)]

    z16 = jnp.zeros((n_pad, 16), _f32)
    z32 = jnp.zeros((n_pad, 32), _f32)

    h = mm_in(x, Wi.T, bi[None, :])
    agg0 = agg1 = skip = None
    for l in range(n_layers):
        wcat = jnp.concatenate([Wq[l].T, Wk[l].T, Wv[l].T, Ws[l].T], axis=1)
        bcat = jnp.concatenate([bq[l], bk[l], bv[l], bs[l]])[None, :]
        if l == 0:
            Q, K, V0, V1, skip = proj_h(h, wcat, bcat)
        else:
            Q, K, V0, V1, skip = proj_agg(agg0, agg1, skip, wcat, bcat)
        ep0, ep1 = edge_proj(edge_attr, We[l].T, be[l][None, :])
        ex, den = p1(Q, K, ep0, ep1, src, dst, z16)
        inv = inv_den(den)
        agg0 = p2[0](V0, ep0, ex, inv, src, dst, z32)
        agg1 = p2[1](V1, ep1, ex, inv, src, dst, z32)
    return final(agg0, agg1, skip, Wo.T, bo[None, :])


# speedup vs baseline: 15.7481x; 1.0424x over previous
import functools
import math

import jax
import jax.numpy as jnp
from jax import lax
from jax.experimental import pallas as pl
from jax.experimental.pallas import tpu as pltpu
from jax.experimental.pallas import tpu_sc as plsc

_NC = 2
_NS = 16
_NW = _NC * _NS
_CB = 128
_HEADS = 8

_SC_PARAMS = pltpu.CompilerParams(use_tc_tiling_on_sc=False,
                                  needs_layout_passes=False)

_f32 = jnp.float32
_i32 = jnp.int32


def _mm_in_body(x_ref, w_ref, b_ref, o_ref):
    o_ref[...] = jnp.dot(x_ref[...], w_ref[...],
                         preferred_element_type=_f32) + b_ref[...]


def _proj_h_body(h_ref, w_ref, b_ref, q_ref, k_ref, v0_ref, v1_ref, s_ref):
    p = jnp.dot(h_ref[...], w_ref[...], preferred_element_type=_f32) + b_ref[...]
    hc = q_ref.shape[1]
    q_ref[...] = p[:, :hc]
    k_ref[...] = p[:, hc:2 * hc]
    v0_ref[...] = p[:, 2 * hc:2 * hc + hc // 2]
    v1_ref[...] = p[:, 2 * hc + hc // 2:3 * hc]
    s_ref[...] = p[:, 3 * hc:]


def _proj_agg_body(a0_ref, a1_ref, sin_ref, w_ref, b_ref,
                   q_ref, k_ref, v0_ref, v1_ref, s_ref):
    h = jnp.concatenate([a0_ref[0] + a0_ref[1], a1_ref[0] + a1_ref[1]],
                        axis=1) + sin_ref[...]
    p = jnp.dot(h, w_ref[...], preferred_element_type=_f32) + b_ref[...]
    hc = q_ref.shape[1]
    q_ref[...] = p[:, :hc]
    k_ref[...] = p[:, hc:2 * hc]
    v0_ref[...] = p[:, 2 * hc:2 * hc + hc // 2]
    v1_ref[...] = p[:, 2 * hc + hc // 2:3 * hc]
    s_ref[...] = p[:, 3 * hc:]


def _edge_body(ea_ref, w_ref, b_ref, e0_ref, e1_ref):
    p = jnp.dot(ea_ref[...], w_ref[...], preferred_element_type=_f32) + b_ref[...]
    hf = e0_ref.shape[1]
    e0_ref[...] = p[:, :hf]
    e1_ref[...] = p[:, hf:]


def _inv_body(den_ref, o_ref):
    o_ref[...] = 1.0 / (den_ref[0] + den_ref[1] + 1e-16)


def _final_body(a0_ref, a1_ref, sin_ref, w_ref, b_ref, o_ref):
    h = jnp.concatenate([a0_ref[0] + a0_ref[1], a1_ref[0] + a1_ref[1]],
                        axis=1) + sin_ref[...]
    o_ref[...] = jax.nn.sigmoid(
        jnp.dot(h, w_ref[...], preferred_element_type=_f32) + b_ref[...])


def _row_spec(br, w):
    return pl.BlockSpec((br, w), lambda i: (i, 0))


def _fix_spec(shape):
    nd = len(shape)
    return pl.BlockSpec(shape, lambda i: (0,) * nd)


def _pair_spec(br, w):
    return pl.BlockSpec((2, br, w), lambda i: (0, i, 0))


def _pass1_body(scale, nblk, q_h, k_h, ep0_h, ep1_h, ei_h, z_h,
                ex_h, den_h,
                idx_sd, qb, kb, eb0, eb1, exb, den_sh):
    c = lax.axis_index("c")
    s = lax.axis_index("s")
    wid = s * _NC + c
    n = den_sh.shape[0]
    rp = n // _NS
    r0 = s * rp
    pltpu.sync_copy(z_h.at[pl.ds(r0, rp)], den_sh.at[pl.ds(r0, rp)])
    zv = jnp.zeros((16,), _f32)

    def _zb(i, _):
        exb[i] = zv
        return 0
    lax.fori_loop(0, _CB, _zb, 0)
    plsc.subcore_barrier()

    iota = lax.iota(_i32, 16)
    hc = qb.shape[1]
    nloop = (nblk + _NW - 1) // _NW

    def _blk(i, _):
        blk = wid + i * _NW

        @pl.when(blk < nblk)
        def _():
            e0 = blk * _CB
            pltpu.sync_copy(ei_h.at[:, pl.ds(e0, _CB)], idx_sd)
            pltpu.sync_copy(q_h.at[idx_sd.at[1]], qb)
            pltpu.sync_copy(k_h.at[idx_sd.at[0]], kb)
            pltpu.sync_copy(ep0_h.at[pl.ds(e0, _CB)], eb0)
            pltpu.sync_copy(ep1_h.at[pl.ds(e0, _CB)], eb1)

            def _grp(g, _2):
                erow = iota + g * 16
                acc = [jnp.zeros((16,), _f32)] * _HEADS
                hw = hc // _HEADS
                for cf in range(hc):
                    col = jnp.full((16,), cf, _i32)
                    qv = plsc.load_gather(qb, [erow, col])
                    kv = plsc.load_gather(kb, [erow, col])
                    if cf < hc // 2:
                        ev = plsc.load_gather(eb0, [erow, col])
                    else:
                        ev = plsc.load_gather(
                            eb1, [erow, jnp.full((16,), cf - hc // 2, _i32)])
                    hh = cf // hw
                    acc[hh] = acc[hh] + qv * (kv + ev)
                for hh in range(_HEADS):
                    exv = jnp.exp(acc[hh] * scale)
                    plsc.store_scatter(exb, [erow, jnp.full((16,), hh, _i32)], exv)
                return 0
            lax.fori_loop(0, _CB // 16, _grp, 0)

            pltpu.sync_copy(exb, ex_h.at[pl.ds(e0, _CB)])
            pltpu.sync_copy(exb, den_sh.at[idx_sd.at[1]], add=True)
        return 0
    lax.fori_loop(0, nloop, _blk, 0)

    plsc.subcore_barrier()
    pltpu.sync_copy(den_sh.at[pl.ds(r0, rp)], den_h.at[c, pl.ds(r0, rp)])


def _pass2_body(hh, nblk, v_h, ep_h, ex_h, inv_h, ei_h, z_h,
                agg_h,
                idx_sd, vb, eb, exb, ivb, mb, agg_sh):
    c = lax.axis_index("c")
    s = lax.axis_index("s")
    wid = s * _NC + c
    n = agg_sh.shape[0]
    rp = n // _NS
    r0 = s * rp
    pltpu.sync_copy(z_h.at[pl.ds(r0, rp)], agg_sh.at[pl.ds(r0, rp)])
    plsc.subcore_barrier()

    iota = lax.iota(_i32, 16)
    sh3 = lax.shift_right_logical(iota, 3)
    col0 = jnp.full((16,), 4 * hh, _i32) + sh3
    col1 = jnp.full((16,), 4 * hh + 2, _i32) + sh3
    nloop = (nblk + _NW - 1) // _NW

    def _blk(i, _):
        blk = wid + i * _NW

        @pl.when(blk < nblk)
        def _():
            e0 = blk * _CB
            pltpu.sync_copy(ei_h.at[:, pl.ds(e0, _CB)], idx_sd)
            pltpu.sync_copy(v_h.at[idx_sd.at[0]], vb)
            pltpu.sync_copy(inv_h.at[idx_sd.at[1]], ivb)
            pltpu.sync_copy(ep_h.at[pl.ds(e0, _CB)], eb)
            pltpu.sync_copy(ex_h.at[pl.ds(e0, _CB)], exb)

            def _edge(e, _2):
                se = jnp.full((16,), e, _i32)
                a0 = (plsc.load_gather(exb, [se, col0]) *
                      plsc.load_gather(ivb, [se, col0]))
                a1 = (plsc.load_gather(exb, [se, col1]) *
                      plsc.load_gather(ivb, [se, col1]))
                mb[e, pl.ds(0, 16)] = (vb[e, pl.ds(0, 16)] +
                                       eb[e, pl.ds(0, 16)]) * a0
                mb[e, pl.ds(16, 16)] = (vb[e, pl.ds(16, 16)] +
                                        eb[e, pl.ds(16, 16)]) * a1
                return 0
            lax.fori_loop(0, _CB, _edge, 0)
            pltpu.sync_copy(mb, agg_sh.at[idx_sd.at[1]], add=True)
        return 0
    lax.fori_loop(0, nloop, _blk, 0)

    plsc.subcore_barrier()
    pltpu.sync_copy(agg_sh.at[pl.ds(r0, rp)], agg_h.at[c, pl.ds(r0, rp)])


def _build_pass1(n_pad, e, hc, scale):
    nblk = e // _CB
    mesh = plsc.VectorSubcoreMesh(core_axis_name="c", subcore_axis_name="s")
    return pl.kernel(
        functools.partial(_pass1_body, scale, nblk),
        compiler_params=_SC_PARAMS,
        out_type=(jax.ShapeDtypeStruct((e, 16), _f32),
                  jax.ShapeDtypeStruct((2, n_pad, 16), _f32)),
        mesh=mesh,
        scratch_types=[
            pltpu.VMEM((2, _CB), _i32),
            pltpu.VMEM((_CB, hc), _f32),
            pltpu.VMEM((_CB, hc), _f32),
            pltpu.VMEM((_CB, hc // 2), _f32),
            pltpu.VMEM((_CB, hc // 2), _f32),
            pltpu.VMEM((_CB, 16), _f32),
            pltpu.VMEM_SHARED((n_pad, 16), _f32),
        ],
    )


def _build_pass2(n_pad, e, hc, hh):
    nblk = e // _CB
    hf = hc // 2
    mesh = plsc.VectorSubcoreMesh(core_axis_name="c", subcore_axis_name="s")
    return pl.kernel(
        functools.partial(_pass2_body, hh, nblk),
        compiler_params=_SC_PARAMS,
        out_type=jax.ShapeDtypeStruct((2, n_pad, hf), _f32),
        mesh=mesh,
        scratch_types=[
            pltpu.VMEM((2, _CB), _i32),
            pltpu.VMEM((_CB, hf), _f32),
            pltpu.VMEM((_CB, hf), _f32),
            pltpu.VMEM((_CB, 16), _f32),
            pltpu.VMEM((_CB, 16), _f32),
            pltpu.VMEM((_CB, hf), _f32),
            pltpu.VMEM_SHARED((n_pad, hf), _f32),
        ],
    )


def kernel(x, edge_index, edge_attr, Wi, bi, Wq, bq, Wk, bk, Wv, bv,
           We, be, Ws, bs, Wo, bo):
    n, d_in = x.shape
    e = edge_index.shape[1]
    hc = Wi.shape[0]
    d_e = edge_attr.shape[1]
    n_layers = Wq.shape[0]
    hid = hc // _HEADS
    scale = 1.0 / math.sqrt(hid)
    br = 2000
    br_e = 4000

    mm_in = pl.pallas_call(
        _mm_in_body,
        grid=(n // br,),
        in_specs=[_row_spec(br, d_in), _fix_spec((d_in, hc)), _fix_spec((1, hc))],
        out_specs=_row_spec(br, hc),
        out_shape=jax.ShapeDtypeStruct((n, hc), _f32),
    )
    proj_outs = dict(
        out_specs=[_row_spec(br, hc), _row_spec(br, hc),
                   _row_spec(br, hc // 2), _row_spec(br, hc // 2),
                   _row_spec(br, hc)],
        out_shape=[jax.ShapeDtypeStruct((n, hc), _f32),
                   jax.ShapeDtypeStruct((n, hc), _f32),
                   jax.ShapeDtypeStruct((n, hc // 2), _f32),
                   jax.ShapeDtypeStruct((n, hc // 2), _f32),
                   jax.ShapeDtypeStruct((n, hc), _f32)],
    )
    proj_h = pl.pallas_call(
        _proj_h_body,
        grid=(n // br,),
        in_specs=[_row_spec(br, hc), _fix_spec((hc, 4 * hc)), _fix_spec((1, 4 * hc))],
        **proj_outs,
    )
    proj_agg = pl.pallas_call(
        _proj_agg_body,
        grid=(n // br,),
        in_specs=[_pair_spec(br, hc // 2), _pair_spec(br, hc // 2),
                  _row_spec(br, hc), _fix_spec((hc, 4 * hc)), _fix_spec((1, 4 * hc))],
        **proj_outs,
    )
    edge_proj = pl.pallas_call(
        _edge_body,
        grid=(e // br_e,),
        in_specs=[_row_spec(br_e, d_e), _fix_spec((d_e, hc)), _fix_spec((1, hc))],
        out_specs=[_row_spec(br_e, hc // 2), _row_spec(br_e, hc // 2)],
        out_shape=[jax.ShapeDtypeStruct((e, hc // 2), _f32),
                   jax.ShapeDtypeStruct((e, hc // 2), _f32)],
    )
    final = pl.pallas_call(
        _final_body,
        grid=(n // br,),
        in_specs=[_pair_spec(br, hc // 2), _pair_spec(br, hc // 2),
                  _row_spec(br, hc), _fix_spec((hc, 1)), _fix_spec((1, 1))],
        out_specs=_row_spec(br, 1),
        out_shape=jax.ShapeDtypeStruct((n, 1), _f32),
    )

    n_pad = -(-n // 128) * 128
    br_i = n_pad // 8
    inv_den = pl.pallas_call(
        _inv_body,
        grid=(n_pad // br_i,),
        in_specs=[_pair_spec(br_i, 16)],
        out_specs=_row_spec(br_i, 16),
        out_shape=jax.ShapeDtypeStruct((n_pad, 16), _f32),
    )

    p1 = _build_pass1(n_pad, e, hc, scale)
    p2 = [_build_pass2(n_pad, e, hc, 0), _build_pass2(n_pad, e, hc, 1)]

    z16 = jnp.zeros((n_pad, 16), _f32)
    z32 = jnp.zeros((n_pad, 32), _f32)

    h = mm_in(x, Wi.T, bi[None, :])
    agg0 = agg1 = skip = None
    for l in range(n_layers):
        wcat = jnp.concatenate([Wq[l].T, Wk[l].T, Wv[l].T, Ws[l].T], axis=1)
        bcat = jnp.concatenate([bq[l], bk[l], bv[l], bs[l]])[None, :]
        if l == 0:
            Q, K, V0, V1, skip = proj_h(h, wcat, bcat)
        else:
            Q, K, V0, V1, skip = proj_agg(agg0, agg1, skip, wcat, bcat)
        ep0, ep1 = edge_proj(edge_attr, We[l].T, be[l][None, :])
        ex, den = p1(Q, K, ep0, ep1, edge_index, z16)
        inv = inv_den(den)
        agg0 = p2[0](V0, ep0, ex, inv, edge_index, z32)
        agg1 = p2[1](V1, ep1, ex, inv, edge_index, z32)
    return final(agg0, agg1, skip, Wo.T, bo[None, :])
